# Initial kernel scaffold; baseline (speedup 1.0000x reference)
import jax
import jax.numpy as jnp
from jax.experimental import pallas as pl


def kernel(x, dijk, idx_j, seg_i, seg_j, seg_i_sum, W1, b1, W2, b2, Win, Wout, bout, Wd, bd):
    raise NotImplementedError("write your pallas kernel here")


# speedup vs baseline: 3.5401x; 3.5401x over previous
import functools

import jax
import jax.numpy as jnp
from jax import lax
from jax.experimental import pallas as pl
from jax.experimental.pallas import tpu as pltpu
from jax.experimental.pallas import tpu_sc as plsc

N = 10000
E = 160000
F = 128

_LOG2 = 0.6931471805599453


def _ssp(z):
    return jnp.maximum(z, 0.0) + jnp.log1p(jnp.exp(-jnp.abs(z))) - _LOG2


_BE = 1600


def _filter_body(dijk_ref, w1_ref, b1_ref, w2_ref, b2_ref, out_ref):
    h = jnp.dot(dijk_ref[...], w1_ref[...], preferred_element_type=jnp.float32)
    h = _ssp(h + b1_ref[...])
    w = jnp.dot(h, w2_ref[...], preferred_element_type=jnp.float32)
    out_ref[...] = _ssp(w + b2_ref[...])


def _filter(dijk, W1, b1, W2, b2):
    return pl.pallas_call(
        _filter_body,
        grid=(E // _BE,),
        in_specs=[
            pl.BlockSpec((_BE, F), lambda i: (i, 0)),
            pl.BlockSpec((F, F), lambda i: (0, 0)),
            pl.BlockSpec((1, F), lambda i: (0, 0)),
            pl.BlockSpec((F, F), lambda i: (0, 0)),
            pl.BlockSpec((1, F), lambda i: (0, 0)),
        ],
        out_specs=pl.BlockSpec((_BE, F), lambda i: (i, 0)),
        out_shape=jax.ShapeDtypeStruct((E, F), jnp.float32),
    )(dijk, W1, b1.reshape(1, F), W2, b2.reshape(1, F))


_BN = 1000


def _in2fac_body(x_ref, win_ref, f_ref):
    f_ref[...] = jnp.dot(x_ref[...], win_ref[...],
                         preferred_element_type=jnp.float32)


def _in2fac(x, Win):
    return pl.pallas_call(
        _in2fac_body,
        grid=(N // _BN,),
        in_specs=[
            pl.BlockSpec((_BN, F), lambda i: (i, 0)),
            pl.BlockSpec((F, F), lambda i: (0, 0)),
        ],
        out_specs=pl.BlockSpec((_BN, F), lambda i: (i, 0)),
        out_shape=jax.ShapeDtypeStruct((N, F), jnp.float32),
    )(x, Win)


_NC = 2
_NS = 16
_NW = _NC * _NS
_CHUNK = 128
_NCHUNK = E // _CHUNK
_STEPS = (_NCHUNK + _NW - 1) // _NW
_NPAD = 10240
_ROWS_PER_TILE = _NPAD // _NS


def _sc_conv_body(f_hbm, w_hbm, idx_hbm, seg_hbm, zeros_hbm, out_hbm,
                  idx_v, seg_v, rows_v, w_v, conv_sh, sem_g, sem_w):
    cid = lax.axis_index("c")
    sid = lax.axis_index("s")
    wid = cid * _NS + sid

    pltpu.sync_copy(zeros_hbm.at[pl.ds(sid * _ROWS_PER_TILE, _ROWS_PER_TILE)],
                    conv_sh.at[pl.ds(sid * _ROWS_PER_TILE, _ROWS_PER_TILE)])
    plsc.subcore_barrier()

    def step(k, _):
        chunk = k * _NW + wid

        @pl.when(chunk < _NCHUNK)
        def _():
            base = chunk * _CHUNK
            pltpu.sync_copy(idx_hbm.at[pl.ds(base, _CHUNK)], idx_v)
            pltpu.sync_copy(seg_hbm.at[pl.ds(base, _CHUNK)], seg_v)
            cp_g = pltpu.async_copy(f_hbm.at[idx_v], rows_v, sem_g)
            cp_w = pltpu.async_copy(w_hbm.at[pl.ds(base, _CHUNK)], w_v, sem_w)
            cp_g.wait()
            cp_w.wait()

            def mul_row(e, _):
                for c in range(F // 16):
                    sl = pl.ds(c * 16, 16)
                    rows_v[e, sl] = rows_v[e, sl] * w_v[e, sl]
                return 0

            lax.fori_loop(0, _CHUNK, mul_row, 0)
            pltpu.sync_copy(rows_v, conv_sh.at[seg_v], add=True)

        return 0

    lax.fori_loop(0, _STEPS, step, 0)
    plsc.subcore_barrier()

    off = sid * _ROWS_PER_TILE
    pltpu.sync_copy(conv_sh.at[pl.ds(off, _ROWS_PER_TILE)],
                    out_hbm.at[cid, pl.ds(off, _ROWS_PER_TILE)])


def _sc_conv(f, w, idx_j, seg_i, zeros):
    mesh = plsc.VectorSubcoreMesh(core_axis_name="c", subcore_axis_name="s")
    kern = functools.partial(
        pl.kernel,
        out_type=jax.ShapeDtypeStruct((_NC, _NPAD, F), jnp.float32),
        mesh=mesh,
        scratch_types=[
            pltpu.VMEM((_CHUNK,), jnp.int32),
            pltpu.VMEM((_CHUNK,), jnp.int32),
            pltpu.VMEM((_CHUNK, F), jnp.float32),
            pltpu.VMEM((_CHUNK, F), jnp.float32),
            pltpu.VMEM_SHARED((_NPAD, F), jnp.float32),
            pltpu.SemaphoreType.DMA,
            pltpu.SemaphoreType.DMA,
        ],
    )(_sc_conv_body)
    return kern(f, w, idx_j, seg_i, zeros)


def _epilogue_body(p0_ref, p1_ref, x_ref, wout_ref, bout_ref, wd_ref, bd_ref,
                   y_ref, v_ref):
    conv = p0_ref[0] + p1_ref[0]
    c = _ssp(jnp.dot(conv, wout_ref[...], preferred_element_type=jnp.float32)
             + bout_ref[...])
    v = jnp.dot(c, wd_ref[...], preferred_element_type=jnp.float32) + bd_ref[...]
    v_ref[...] = v
    y_ref[...] = x_ref[...] + v


def _epilogue(parts, x, Wout, bout, Wd, bd):
    nb = N // _BN
    return pl.pallas_call(
        _epilogue_body,
        grid=(nb,),
        in_specs=[
            pl.BlockSpec((1, _BN, F), lambda i: (0, i, 0)),
            pl.BlockSpec((1, _BN, F), lambda i: (1, i, 0)),
            pl.BlockSpec((_BN, F), lambda i: (i, 0)),
            pl.BlockSpec((F, F), lambda i: (0, 0)),
            pl.BlockSpec((1, F), lambda i: (0, 0)),
            pl.BlockSpec((F, F), lambda i: (0, 0)),
            pl.BlockSpec((1, F), lambda i: (0, 0)),
        ],
        out_specs=[
            pl.BlockSpec((_BN, F), lambda i: (i, 0)),
            pl.BlockSpec((_BN, F), lambda i: (i, 0)),
        ],
        out_shape=[
            jax.ShapeDtypeStruct((N, F), jnp.float32),
            jax.ShapeDtypeStruct((N, F), jnp.float32),
        ],
    )(parts, parts, x, Wout, bout.reshape(1, F), Wd, bd.reshape(1, F))


def kernel(x, dijk, idx_j, seg_i, seg_j, seg_i_sum,
           W1, b1, W2, b2, Win, Wout, bout, Wd, bd):
    w = _filter(dijk, W1, b1, W2, b2)
    f = _in2fac(x, Win)
    zeros = jnp.zeros((_NPAD, F), jnp.float32)
    parts = _sc_conv(f, w, idx_j.astype(jnp.int32), seg_i.astype(jnp.int32),
                     zeros)
    y, v = _epilogue(parts, x, Wout, bout, Wd, bd)
    return (y, v)
